# baseline (speedup 1.0000x reference)
import functools
import numpy as np
import jax
import jax.numpy as jnp
from jax import lax
from jax.experimental import pallas as pl
from jax.experimental.pallas import tpu as pltpu
from jax.experimental.pallas import tpu_sc as plsc

_RES3 = [18, 24, 33, 44, 59, 80, 108, 148, 201, 275, 376, 514]
_RES2 = [130, 258, 514, 1026]
_H3 = 2 ** 19
_H2 = 2 ** 17
_NF = 4
_P1 = int(np.array(2654435761, np.uint32).view(np.int32))
_P2 = 805459861

_NC, _NS = 2, 16
_NW = _NC * _NS
_B = 64
_NSUB = _B // 16

_NL3 = len(_RES3)
_NL2 = len(_RES2)
_F3 = _NL3 * _NF
_FT = _F3 + 3 * _NL2 * _NF

_R3 = _NL3 * 8 * _B // 128
_R2 = 3 * _NL2 * 4 * _B // 128


def _levels(res_list, hmax, dim):
    out = []
    off = 0
    for r in res_list:
        direct = (r + 1) ** dim <= hmax
        size = (r + 1) ** dim if direct else hmax
        out.append((r, direct, off))
        off += size
    return out


_LV3 = _levels(_RES3, _H3, 3)
_LV2 = _levels(_RES2, _H2, 2)


def _split_dim(pos):
    pf = pos.astype(jnp.int32)
    fr = pos - pf.astype(jnp.float32)
    return pf, fr, 1.0 - fr


def _body(xT, t3, txy, txz, tyz, out, xv, idx3, g3, w3, idx2a, idx2b, idx2c,
          g2, w2, outv, sem, *, np_pts, nblk):
    wid = lax.axis_index("s") * _NC + lax.axis_index("c")
    pbase = wid * np_pts

    for d in range(3):
        pltpu.sync_copy(xT.at[d, pl.ds(pbase, np_pts)], xv.at[d])

    iota = lax.iota(jnp.int32, 16)
    qdiv4 = iota >> 2
    fmod = iota & 3
    oc_off = qdiv4 * _FT + fmod

    @pl.loop(0, nblk)
    def _block(blk):
        bb = blk * _B

        @pl.loop(0, _NSUB)
        def _phase_a(i):
            col16 = 16 * i
            xs = [xv[d, pl.ds(bb + col16, 16)] for d in range(3)]

            for l, (r, direct, off) in enumerate(_LV3):
                rf = float(r)
                pf0, f0, g0 = _split_dim(xs[0] * rf)
                pf1, f1, g1 = _split_dim(xs[1] * rf)
                pf2, f2, g2_ = _split_dim(xs[2] * rf)
                if direct:
                    s1 = r + 1
                    s2 = s1 * s1
                    a = (pf0, pf0 + 1)
                    b = (pf1 * s1, pf1 * s1 + s1)
                    c = (pf2 * s2 + off, pf2 * s2 + s2 + off)
                else:
                    a = (pf0, pf0 + 1)
                    b = (pf1 * _P1, pf1 * _P1 + _P1)
                    c = (pf2 * _P2, pf2 * _P2 + _P2)
                w01 = [[g0 * g1, g0 * f1], [f0 * g1, f0 * f1]]
                t2 = (g2_, f2)
                for cid in range(8):
                    b0, b1, b2 = cid >> 2, (cid >> 1) & 1, cid & 1
                    if direct:
                        idx = a[b0] + b[b1] + c[b2]
                    else:
                        idx = ((a[b0] ^ b[b1] ^ c[b2]) & (_H3 - 1)) + off
                    w = w01[b0][b1] * t2[b2]
                    fl = l * 512 + cid * 64
                    row = fl // 128
                    colbase = fl % 128
                    idx3[row, pl.ds(colbase + col16, 16)] = idx
                    w3[pl.ds(fl + col16, 16)] = w

            for p, (da, db) in enumerate(((0, 1), (0, 2), (1, 2))):
                for l, (r, direct, off) in enumerate(_LV2):
                    rf = float(r)
                    pfa, fa, ga = _split_dim(xs[da] * rf)
                    pfb, fb, gb = _split_dim(xs[db] * rf)
                    if direct:
                        s1 = r + 1
                        a = (pfa + off, pfa + 1 + off)
                        b = (pfb * s1, pfb * s1 + s1)
                    else:
                        a = (pfa, pfa + 1)
                        b = (pfb * _P1, pfb * _P1 + _P1)
                    wa = (ga, fa)
                    wb = (gb, fb)
                    for cid in range(4):
                        b0, b1 = cid >> 1, cid & 1
                        if direct:
                            idx = a[b0] + b[b1]
                        else:
                            idx = ((a[b0] ^ b[b1]) & (_H2 - 1)) + off
                        w = wa[b0] * wb[b1]
                        fl = (p * _NL2 + l) * 256 + cid * 64
                        row = (l * 256 + cid * 64) // 128
                        colbase = fl % 128
                        idx2p = (idx2a, idx2b, idx2c)[p]
                        idx2p[row, pl.ds(colbase + col16, 16)] = idx
                        w2[pl.ds(fl + col16, 16)] = w

        @pl.loop(0, _R3)
        def _fire3(j):
            pltpu.async_copy(t3.at[idx3.at[j]], g3.at[j], sem)

        for p, tbl in enumerate((txy, txz, tyz)):
            idx2p = (idx2a, idx2b, idx2c)[p]

            @pl.loop(0, _R2 // 3)
            def _fire2(j, tbl=tbl, idx2p=idx2p, p=p):
                pltpu.async_copy(tbl.at[idx2p.at[j]], g2.at[p, j], sem)

        @pl.loop(0, _R3)
        def _wait3(j):
            pltpu.make_async_copy(t3.at[idx3.at[j]], g3.at[j], sem).wait()

        for p, tbl in enumerate((txy, txz, tyz)):
            idx2p = (idx2a, idx2b, idx2c)[p]

            @pl.loop(0, _R2 // 3)
            def _wait2(j, tbl=tbl, idx2p=idx2p, p=p):
                pltpu.make_async_copy(tbl.at[idx2p.at[j]], g2.at[p, j],
                                      sem).wait()

        @pl.loop(0, _NSUB)
        def _phase_b(i):
            col16 = 16 * i
            z = jnp.zeros(16, jnp.float32)

            @pl.loop(0, _NL3)
            def _lvl3(l):
                def corner(c, accs):
                    fl = l * 512 + c * 64 + col16
                    new = []
                    for q in range(4):
                        t = qdiv4 + (fl + 4 * q)
                        feat = plsc.load_gather(g3, [t >> 7, t & 127, fmod])
                        wv = plsc.load_gather(w3, [t])
                        new.append(accs[q] + feat * wv)
                    return tuple(new)

                accs = pl.loop(0, 8, init_carry=(z, z, z, z))(corner)
                for q in range(4):
                    oidx = oc_off + ((col16 + 4 * q) * _FT + 4 * l)
                    plsc.store_scatter(outv, [oidx], accs[q])

            for p in range(3):
                @pl.loop(0, _NL2)
                def _lvl2(l, p=p):
                    def corner(c, accs):
                        flp = l * 256 + c * 64 + col16
                        new = []
                        for q in range(4):
                            t = qdiv4 + (flp + 4 * q)
                            feat = plsc.load_gather(
                                g2.at[p], [t >> 7, t & 127, fmod])
                            wv = plsc.load_gather(w2, [t + p * 1024])
                            new.append(accs[q] + feat * wv)
                        return tuple(new)

                    accs = pl.loop(0, 4, init_carry=(z, z, z, z))(corner)
                    for q in range(4):
                        oidx = oc_off + ((col16 + 4 * q) * _FT + _F3
                                         + 4 * (p * _NL2 + l))
                        plsc.store_scatter(outv, [oidx], accs[q])

        pltpu.sync_copy(outv, out.at[pl.ds((pbase + bb) * _FT, _B * _FT)])


@jax.jit
def _encode(xT, t3, txy, txz, tyz):
    n = xT.shape[1]
    np_pts = n // _NW
    nblk = np_pts // _B
    mesh = plsc.VectorSubcoreMesh(core_axis_name="c", subcore_axis_name="s",
                                  num_cores=_NC, num_subcores=_NS)
    body = functools.partial(_body, np_pts=np_pts, nblk=nblk)
    return pl.kernel(
        body,
        out_type=jax.ShapeDtypeStruct((n * _FT,), jnp.float32),
        mesh=mesh,
        compiler_params=pltpu.CompilerParams(use_tc_tiling_on_sc=False,
                                             needs_layout_passes=False),
        scratch_types=[
            pltpu.VMEM((3, np_pts), jnp.float32),
            pltpu.VMEM((_R3, 128), jnp.int32),
            pltpu.VMEM((_R3, 128, 2 * _NF), jnp.float32),
            pltpu.VMEM((_R3 * 128,), jnp.float32),
            pltpu.VMEM((_R2 // 3, 128), jnp.int32),
            pltpu.VMEM((_R2 // 3, 128), jnp.int32),
            pltpu.VMEM((_R2 // 3, 128), jnp.int32),
            pltpu.VMEM((3, _R2 // 3, 128, 2 * _NF), jnp.float32),
            pltpu.VMEM((_R2 * 128,), jnp.float32),
            pltpu.VMEM((_B * _FT,), jnp.float32),
            pltpu.SemaphoreType.DMA,
        ],
    )(xT, t3, txy, txz, tyz)


def kernel(x, table_xyz, table_xy, table_xz, table_yz):
    pad = ((0, 0), (0, _NF))
    out = _encode(x.T, jnp.pad(table_xyz, pad), jnp.pad(table_xy, pad),
                  jnp.pad(table_xz, pad), jnp.pad(table_yz, pad))
    return out.reshape(x.shape[0], _FT)


# speedup vs baseline: 2.3653x; 2.3653x over previous
import functools
import numpy as np
import jax
import jax.numpy as jnp
from jax import lax
from jax.experimental import pallas as pl
from jax.experimental.pallas import tpu as pltpu
from jax.experimental.pallas import tpu_sc as plsc

_RES3 = [18, 24, 33, 44, 59, 80, 108, 148, 201, 275, 376, 514]
_RES2 = [130, 258, 514, 1026]
_H3 = 2 ** 19
_H2 = 2 ** 17
_NF = 4
_P1 = int(np.array(2654435761, np.uint32).view(np.int32))
_P2 = 805459861

_NC, _NS = 2, 16
_NW = _NC * _NS
_B = 64
_NSUB = _B // 16

_NL3 = len(_RES3)
_NL2 = len(_RES2)
_F3 = _NL3 * _NF
_FT = _F3 + 3 * _NL2 * _NF

_R3 = _NL3 * 8 * _B // 128
_R2P = _NL2 * 4 * _B // 128


def _levels(res_list, hmax, dim):
    out = []
    off = 0
    for r in res_list:
        direct = (r + 1) ** dim <= hmax
        size = (r + 1) ** dim if direct else hmax
        out.append((r, direct, off))
        off += size
    return out


_LV3 = _levels(_RES3, _H3, 3)
_LV2 = _levels(_RES2, _H2, 2)


def _split_dim(pos):
    pf = pos.astype(jnp.int32)
    fr = pos - pf.astype(jnp.float32)
    return pf, fr, 1.0 - fr


def _body(xT, *args, np_pts, nblk):
    c3 = args[0:4]
    c2 = (args[4:8], args[8:12], args[12:16])
    out = args[16]
    (xv, idx3, g3, w3, idx2a, idx2b, idx2c, g2, w2, outv, sem) = args[17:]
    wid = lax.axis_index("s") * _NC + lax.axis_index("c")
    pbase = wid * np_pts

    for d in range(3):
        pltpu.sync_copy(xT.at[d, pl.ds(pbase, np_pts)], xv.at[d])

    iota = lax.iota(jnp.int32, 16)
    oc96 = iota * _FT

    def phase_a(bb, par):
        @pl.loop(0, _NSUB)
        def _pa(i):
            col16 = 16 * i
            xs = [xv[d, pl.ds(bb + col16, 16)] for d in range(3)]

            for l, (r, direct, off) in enumerate(_LV3):
                rf = float(r)
                pf0, f0, g0 = _split_dim(xs[0] * rf)
                pf1, f1, g1 = _split_dim(xs[1] * rf)
                pf2, f2, g2_ = _split_dim(xs[2] * rf)
                if direct:
                    s1 = r + 1
                    s2 = s1 * s1
                    a = (pf0, pf0 + 1)
                    b = (pf1 * s1, pf1 * s1 + s1)
                    c = (pf2 * s2 + off, pf2 * s2 + s2 + off)
                else:
                    a = (pf0, pf0 + 1)
                    b = (pf1 * _P1, pf1 * _P1 + _P1)
                    c = (pf2 * _P2, pf2 * _P2 + _P2)
                w01 = [[g0 * g1, g0 * f1], [f0 * g1, f0 * f1]]
                t2 = (g2_, f2)
                for cid in range(8):
                    b0, b1, b2 = cid >> 2, (cid >> 1) & 1, cid & 1
                    if direct:
                        idx = a[b0] + b[b1] + c[b2]
                    else:
                        idx = ((a[b0] ^ b[b1] ^ c[b2]) & (_H3 - 1)) + off
                    w = w01[b0][b1] * t2[b2]
                    fl = l * 512 + cid * 64
                    idx3[par, fl // 128, pl.ds(fl % 128 + col16, 16)] = idx
                    w3[par, pl.ds(fl + col16, 16)] = w

            for p, (da, db) in enumerate(((0, 1), (0, 2), (1, 2))):
                for l, (r, direct, off) in enumerate(_LV2):
                    rf = float(r)
                    pfa, fa, ga = _split_dim(xs[da] * rf)
                    pfb, fb, gb = _split_dim(xs[db] * rf)
                    if direct:
                        s1 = r + 1
                        a = (pfa + off, pfa + 1 + off)
                        b = (pfb * s1, pfb * s1 + s1)
                    else:
                        a = (pfa, pfa + 1)
                        b = (pfb * _P1, pfb * _P1 + _P1)
                    wa = (ga, fa)
                    wb = (gb, fb)
                    for cid in range(4):
                        b0, b1 = cid >> 1, cid & 1
                        if direct:
                            idx = a[b0] + b[b1]
                        else:
                            idx = ((a[b0] ^ b[b1]) & (_H2 - 1)) + off
                        w = wa[b0] * wb[b1]
                        fl = (p * _NL2 + l) * 256 + cid * 64
                        row = (l * 256 + cid * 64) // 128
                        idx2p = (idx2a, idx2b, idx2c)[p]
                        idx2p[par, row, pl.ds(fl % 128 + col16, 16)] = idx
                        w2[par, pl.ds(fl + col16, 16)] = w

    def fire(par):
        for f in range(_NF):
            @pl.loop(0, _R3)
            def _f3(j, f=f):
                pltpu.async_copy(c3[f].at[idx3.at[par, j]],
                                 g3.at[par, f, j], sem.at[par])

        for p in range(3):
            idx2p = (idx2a, idx2b, idx2c)[p]
            for f in range(_NF):
                @pl.loop(0, _R2P)
                def _f2(j, f=f, p=p, idx2p=idx2p):
                    pltpu.async_copy(c2[p][f].at[idx2p.at[par, j]],
                                     g2.at[par, f, p * _R2P + j],
                                     sem.at[par])

    def wait(par):
        for f in range(_NF):
            @pl.loop(0, _R3)
            def _w3(j, f=f):
                pltpu.make_async_copy(c3[f].at[idx3.at[par, j]],
                                      g3.at[par, f, j], sem.at[par]).wait()

        for p in range(3):
            idx2p = (idx2a, idx2b, idx2c)[p]
            for f in range(_NF):
                @pl.loop(0, _R2P)
                def _w2(j, f=f, p=p, idx2p=idx2p):
                    pltpu.make_async_copy(c2[p][f].at[idx2p.at[par, j]],
                                          g2.at[par, f, p * _R2P + j],
                                          sem.at[par]).wait()

    def phase_b(bb, par):
        @pl.loop(0, _NSUB)
        def _pb(i):
            col16 = 16 * i
            z = jnp.zeros(16, jnp.float32)

            @pl.loop(0, _NL3)
            def _lvl3(l):
                def corner(c, accs):
                    fl = l * 512 + c * 64 + col16
                    w = w3[par, pl.ds(fl, 16)]
                    return tuple(
                        accs[f] + g3[par, f, fl >> 7, pl.ds(fl & 127, 16)]
                        * w for f in range(_NF))

                accs = pl.loop(0, 8, init_carry=(z, z, z, z))(corner)
                for f in range(_NF):
                    oidx = oc96 + (col16 * _FT + 4 * l + f)
                    plsc.store_scatter(outv, [oidx], accs[f])

            for p in range(3):
                @pl.loop(0, _NL2)
                def _lvl2(l, p=p):
                    def corner(c, accs):
                        flp = l * 256 + c * 64 + col16
                        w = w2[par, pl.ds(p * 1024 + flp, 16)]
                        return tuple(
                            accs[f] + g2[par, f, p * _R2P + (flp >> 7),
                                         pl.ds(flp & 127, 16)] * w
                            for f in range(_NF))

                    accs = pl.loop(0, 4, init_carry=(z, z, z, z))(corner)
                    for f in range(_NF):
                        oidx = oc96 + (col16 * _FT + _F3
                                       + 4 * (p * _NL2 + l) + f)
                        plsc.store_scatter(outv, [oidx], accs[f])

        pltpu.sync_copy(outv, out.at[pl.ds((pbase + bb) * _FT, _B * _FT)])

    phase_a(0, 0)
    fire(0)

    @pl.loop(0, nblk)
    def _block(k):
        par = k & 1
        nxt = 1 - par
        knext = jnp.minimum(k + 1, nblk - 1)
        phase_a(knext * _B, nxt)
        fire(nxt)
        wait(par)
        phase_b(k * _B, par)

    wait(nblk & 1)


@jax.jit
def _encode(xT, cols):
    n = xT.shape[1]
    np_pts = n // _NW
    nblk = np_pts // _B
    mesh = plsc.VectorSubcoreMesh(core_axis_name="c", subcore_axis_name="s",
                                  num_cores=_NC, num_subcores=_NS)
    body = functools.partial(_body, np_pts=np_pts, nblk=nblk)
    return pl.kernel(
        body,
        out_type=jax.ShapeDtypeStruct((n * _FT,), jnp.float32),
        mesh=mesh,
        compiler_params=pltpu.CompilerParams(use_tc_tiling_on_sc=False,
                                             needs_layout_passes=False),
        scratch_types=[
            pltpu.VMEM((3, np_pts), jnp.float32),
            pltpu.VMEM((2, _R3, 128), jnp.int32),
            pltpu.VMEM((2, _NF, _R3, 128), jnp.float32),
            pltpu.VMEM((2, _R3 * 128), jnp.float32),
            pltpu.VMEM((2, _R2P, 128), jnp.int32),
            pltpu.VMEM((2, _R2P, 128), jnp.int32),
            pltpu.VMEM((2, _R2P, 128), jnp.int32),
            pltpu.VMEM((2, _NF, 3 * _R2P, 128), jnp.float32),
            pltpu.VMEM((2, 3 * _R2P * 128), jnp.float32),
            pltpu.VMEM((_B * _FT,), jnp.float32),
            pltpu.SemaphoreType.DMA((2,)),
        ],
    )(xT, *cols)


def kernel(x, table_xyz, table_xy, table_xz, table_yz):
    cols = [table_xyz[:, f] for f in range(_NF)]
    for t in (table_xy, table_xz, table_yz):
        cols.extend(t[:, f] for f in range(_NF))
    out = _encode(x.T, tuple(cols))
    return out.reshape(x.shape[0], _FT)
